# Initial kernel scaffold; baseline (speedup 1.0000x reference)
import jax
import jax.numpy as jnp
from jax.experimental import pallas as pl


def kernel(emb1, edge_index1, edge_attr1, emb2, edge_index2, edge_attr2, avg_v, Wn1, We1, Ws1, Wc1, Wn2, We2, Ws2, Wc2, Mmap1, Mmap2, Mcost1, Mcost2):
    raise NotImplementedError("write your pallas kernel here")


# speedup vs baseline: 4.2539x; 4.2539x over previous
import functools

import jax
import jax.numpy as jnp
from jax import lax
from jax.experimental import pallas as pl
from jax.experimental.pallas import tpu as pltpu
from jax.experimental.pallas import tpu_sc as plsc

_N = 10000
_E = 320000
_DN = 128
_DE = 16
_F = 64

_NC = 2
_NS = 16
_NW = _NC * _NS
_C = 80
_CPW = _E // (_NW * _C)
_NPAD = 10240
_RPS = _NPAD // _NS


def _make_sc_spmm(gather):
  mesh = plsc.VectorSubcoreMesh(core_axis_name="c", subcore_axis_name="s")
  out_type = [jax.ShapeDtypeStruct((_NC, _NPAD, _DN), jnp.float32)]
  scratch = [
      pltpu.VMEM_SHARED((_NPAD, _DN), jnp.float32),
      pltpu.VMEM((_CPW, _C), jnp.int32),
      pltpu.VMEM((_C, _DN), jnp.float32),
      pltpu.SemaphoreType.DMA,
  ]
  if gather:
    scratch.append(pltpu.VMEM((_CPW, _C), jnp.int32))

  def body(*refs):
    if gather:
      (x_hbm, srcr, dstr, zeros_hbm,
       out_hbm, acc, dst_idx, rows, sem, src_idx) = refs
    else:
      (x_hbm, dstr, zeros_hbm,
       out_hbm, acc, dst_idx, rows, sem) = refs
    cid = lax.axis_index("c")
    sid = lax.axis_index("s")
    wid = cid * _NS + sid
    pltpu.sync_copy(zeros_hbm, acc.at[pl.ds(sid * _RPS, _RPS)])
    plsc.subcore_barrier()
    pltpu.sync_copy(dstr.at[wid], dst_idx)
    if gather:
      pltpu.sync_copy(srcr.at[wid], src_idx)
    ebase = wid * _CPW * _C

    def chunk(j, carry):
      if gather:
        pltpu.async_copy(x_hbm.at[src_idx.at[j]], rows, sem).wait()
      else:
        pltpu.async_copy(x_hbm.at[pl.ds(ebase + j * _C, _C)], rows, sem).wait()
      pltpu.sync_copy(rows, acc.at[dst_idx.at[j]], add=True)
      return carry

    lax.fori_loop(0, _CPW, chunk, 0)
    plsc.subcore_barrier()
    sl = pl.ds(sid * _RPS, _RPS)
    pltpu.sync_copy(acc.at[sl], out_hbm.at[cid].at[sl])

  return pl.kernel(body, out_type=out_type, mesh=mesh, scratch_types=scratch)


def _pre_body(e1, e2, wn1, mu1, mu2, xwcat, acc):
  pi = pl.program_id(0)

  @pl.when(pi == 0)
  def _():
    acc[...] = jnp.zeros_like(acc)

  acc[0:1, :] += jnp.sum(e1[...], axis=0, keepdims=True)
  acc[1:2, :] += jnp.sum(e2[...], axis=0, keepdims=True)
  xwcat[...] = jnp.concatenate(
      [jnp.dot(e1[...], wn1[...], preferred_element_type=jnp.float32),
       jnp.dot(e2[...], wn1[...], preferred_element_type=jnp.float32)],
      axis=-1)

  @pl.when(pi == pl.num_programs(0) - 1)
  def _():
    mu1[...] = acc[0:1, :] * (1.0 / _N)
    mu2[...] = acc[1:2, :] * (1.0 / _N)


def _ew_body(ea, we12, out):
  out[...] = jnp.dot(ea[...], we12[...], preferred_element_type=jnp.float32)


def _make_conv_body(tw_off, with_next):
  def body(*refs):
    if with_next:
      (s1p, t1p, x1, s2p, t2p, x2, mu1, mu2, ws, wc, wnext,
       h1, h2, sh1, sh2, hwcat, hacc) = refs
    else:
      (s1p, t1p, x1, s2p, t2p, x2, mu1, mu2, ws, wc,
       h1, h2, sh1, sh2, hacc) = refs
    pi = pl.program_id(0)

    def one(sp, tp, x, mu_other, sp_off):
      s = sp[0] + sp[1]
      t = tp[0] + tp[1]
      agg = s[:, sp_off:sp_off + _F] + t[:, tw_off:tw_off + _F]
      agg += jnp.dot(x[...], ws[...], preferred_element_type=jnp.float32)
      ctx = jnp.dot(mu_other, wc[...], preferred_element_type=jnp.float32)
      return jnp.maximum(agg + ctx, 0.0)

    o1 = one(s1p, t1p, x1, mu2[...], 0)
    o2 = one(s2p, t2p, x2, mu1[...], _F)
    h1[...] = o1
    h2[...] = o2
    if with_next:
      hwcat[...] = jnp.concatenate(
          [jnp.dot(o1, wnext[...], preferred_element_type=jnp.float32),
           jnp.dot(o2, wnext[...], preferred_element_type=jnp.float32)],
          axis=-1)

    @pl.when(pi == 0)
    def _():
      hacc[...] = jnp.zeros_like(hacc)

    hacc[0:1, :] += jnp.sum(o1, axis=0, keepdims=True)
    hacc[1:2, :] += jnp.sum(o2, axis=0, keepdims=True)

    @pl.when(pi == pl.num_programs(0) - 1)
    def _():
      sh1[...] = hacc[0:1, :] * (1.0 / _N)
      sh2[...] = hacc[1:2, :] * (1.0 / _N)

  return body


def _match_body(h1, g1, h2t, g2t, mm1, mm2, mc1, mc2, avg,
                mapping, score, pre_ged, sacc):
  pi = pl.program_id(0)
  f32 = jnp.float32
  a = jnp.dot(h1[...], mm1[...], preferred_element_type=f32)
  b = jnp.dot(g1[...], mm2[...], preferred_element_type=f32)
  logits = (jnp.dot(a, h2t[...], preferred_element_type=f32) +
            jnp.dot(b, g2t[...], preferred_element_type=f32))
  m = jnp.max(logits, axis=1, keepdims=True)
  p = jnp.exp(logits - m)
  s = jnp.sum(p, axis=1, keepdims=True)
  mp = p / s
  mapping[...] = mp
  c = jnp.dot(h1[...], mc1[...], preferred_element_type=f32)
  d = jnp.dot(g1[...], mc2[...], preferred_element_type=f32)
  cost = (jnp.dot(c, h2t[...], preferred_element_type=f32) +
          jnp.dot(d, g2t[...], preferred_element_type=f32))
  partial = jnp.sum(mp * cost)
  prev = jnp.where(pi == 0, 0.0, sacc[0])
  tot = prev + partial
  sacc[0] = tot

  @pl.when(pi == pl.num_programs(0) - 1)
  def _():
    sc = 1.0 / (1.0 + jnp.exp(-tot))
    score[0] = sc
    pre_ged[0] = -jnp.log(sc) * avg[0]


@functools.lru_cache(maxsize=None)
def _get_sc_spmm(gather):
  return _make_sc_spmm(gather)


def _sc_spmm_call(x, src, dst):
  z = jnp.zeros((_RPS, _DN), jnp.float32)
  out, = _get_sc_spmm(True)(x, src, dst, z)
  return out


def _sc_easum_call(ew, dst):
  z = jnp.zeros((_RPS, _DN), jnp.float32)
  out, = _get_sc_spmm(False)(ew, dst, z)
  return out


def kernel(emb1, edge_index1, edge_attr1, emb2, edge_index2, edge_attr2,
           avg_v, Wn1, We1, Ws1, Wc1, Wn2, We2, Ws2, Wc2,
           Mmap1, Mmap2, Mcost1, Mcost2):
  f32 = jnp.float32
  bm = 1000
  grid = _N // bm

  mu1, mu2, xwcat = pl.pallas_call(
      _pre_body,
      grid=(grid,),
      in_specs=[pl.BlockSpec((bm, _DN), lambda i: (i, 0)),
                pl.BlockSpec((bm, _DN), lambda i: (i, 0)),
                pl.BlockSpec((_DN, _F), lambda i: (0, 0))],
      out_specs=[pl.BlockSpec((1, _DN), lambda i: (0, 0)),
                 pl.BlockSpec((1, _DN), lambda i: (0, 0)),
                 pl.BlockSpec((bm, _DN), lambda i: (i, 0))],
      out_shape=[jax.ShapeDtypeStruct((1, _DN), f32),
                 jax.ShapeDtypeStruct((1, _DN), f32),
                 jax.ShapeDtypeStruct((_N, _DN), f32)],
      scratch_shapes=[pltpu.VMEM((2, _DN), f32)],
  )(emb1, emb2, Wn1)

  src1 = edge_index1[0].reshape(_NW, _CPW, _C)
  dst1 = edge_index1[1].reshape(_NW, _CPW, _C)
  src2 = edge_index2[0].reshape(_NW, _CPW, _C)
  dst2 = edge_index2[1].reshape(_NW, _CPW, _C)

  we12 = jnp.concatenate([We1, We2], axis=1)
  bme = 8000
  ew_call = pl.pallas_call(
      _ew_body,
      grid=(_E // bme,),
      in_specs=[pl.BlockSpec((bme, _DE), lambda i: (i, 0)),
                pl.BlockSpec((_DE, _DN), lambda i: (0, 0))],
      out_specs=pl.BlockSpec((bme, _DN), lambda i: (i, 0)),
      out_shape=jax.ShapeDtypeStruct((_E, _DN), f32),
  )
  ew1 = ew_call(edge_attr1, we12)
  ew2 = ew_call(edge_attr2, we12)

  s1p = _sc_spmm_call(xwcat, src1, dst1)
  s2p = _sc_spmm_call(xwcat, src2, dst2)
  t1p = _sc_easum_call(ew1, dst1)
  t2p = _sc_easum_call(ew2, dst2)

  def conv_call(body, din, with_next, s1p_, t1p_, x1_, s2p_, t2p_, x2_,
                m1, m2, ws, wc, *wnext):
    return pl.pallas_call(
        body,
        grid=(grid,),
        in_specs=[
            pl.BlockSpec((_NC, bm, _DN), lambda i: (0, i, 0)),
            pl.BlockSpec((_NC, bm, _DN), lambda i: (0, i, 0)),
            pl.BlockSpec((bm, din), lambda i: (i, 0)),
            pl.BlockSpec((_NC, bm, _DN), lambda i: (0, i, 0)),
            pl.BlockSpec((_NC, bm, _DN), lambda i: (0, i, 0)),
            pl.BlockSpec((bm, din), lambda i: (i, 0)),
            pl.BlockSpec((1, din), lambda i: (0, 0)),
            pl.BlockSpec((1, din), lambda i: (0, 0)),
            pl.BlockSpec((din, _F), lambda i: (0, 0)),
            pl.BlockSpec((din, _F), lambda i: (0, 0)),
        ] + ([pl.BlockSpec((_F, _F), lambda i: (0, 0))] if with_next else []),
        out_specs=[pl.BlockSpec((bm, _F), lambda i: (i, 0)),
                   pl.BlockSpec((bm, _F), lambda i: (i, 0)),
                   pl.BlockSpec((1, _F), lambda i: (0, 0)),
                   pl.BlockSpec((1, _F), lambda i: (0, 0))] +
                  ([pl.BlockSpec((bm, _DN), lambda i: (i, 0))]
                   if with_next else []),
        out_shape=[jax.ShapeDtypeStruct((_N, _F), f32),
                   jax.ShapeDtypeStruct((_N, _F), f32),
                   jax.ShapeDtypeStruct((1, _F), f32),
                   jax.ShapeDtypeStruct((1, _F), f32)] +
                  ([jax.ShapeDtypeStruct((_N, _DN), f32)]
                   if with_next else []),
        scratch_shapes=[pltpu.VMEM((2, _F), f32)],
    )(s1p_, t1p_, x1_, s2p_, t2p_, x2_, m1, m2, ws, wc, *wnext)

  h1, h2, mh1, mh2, hwcat = conv_call(
      _make_conv_body(0, True), _DN, True,
      s1p, t1p, emb1, s2p, t2p, emb2, mu1, mu2, Ws1, Wc1, Wn2)

  u1p = _sc_spmm_call(hwcat, src1, dst1)
  u2p = _sc_spmm_call(hwcat, src2, dst2)

  g1, g2, _, _ = conv_call(
      _make_conv_body(_F, False), _F, False,
      u1p, t1p, h1, u2p, t2p, h2, mh1, mh2, Ws2, Wc2)

  bm2 = 200
  grid2 = _N // bm2
  h2t = h2.T
  g2t = g2.T
  mapping, score, pre_ged = pl.pallas_call(
      _match_body,
      grid=(grid2,),
      in_specs=[
          pl.BlockSpec((bm2, _F), lambda i: (i, 0)),
          pl.BlockSpec((bm2, _F), lambda i: (i, 0)),
          pl.BlockSpec((_F, _N), lambda i: (0, 0)),
          pl.BlockSpec((_F, _N), lambda i: (0, 0)),
          pl.BlockSpec((_F, _F), lambda i: (0, 0)),
          pl.BlockSpec((_F, _F), lambda i: (0, 0)),
          pl.BlockSpec((_F, _F), lambda i: (0, 0)),
          pl.BlockSpec((_F, _F), lambda i: (0, 0)),
          pl.BlockSpec(memory_space=pltpu.SMEM),
      ],
      out_specs=[
          pl.BlockSpec((bm2, _N), lambda i: (i, 0)),
          pl.BlockSpec(memory_space=pltpu.SMEM),
          pl.BlockSpec(memory_space=pltpu.SMEM),
      ],
      out_shape=[jax.ShapeDtypeStruct((_N, _N), f32),
                 jax.ShapeDtypeStruct((1,), f32),
                 jax.ShapeDtypeStruct((1,), f32)],
      scratch_shapes=[pltpu.SMEM((1,), f32)],
  )(h1, g1, h2t, g2t, Mmap1, Mmap2, Mcost1, Mcost2, avg_v)

  return score, pre_ged, mapping


# speedup vs baseline: 5.5422x; 1.3029x over previous
import functools

import jax
import jax.numpy as jnp
from jax import lax
from jax.experimental import pallas as pl
from jax.experimental.pallas import tpu as pltpu
from jax.experimental.pallas import tpu_sc as plsc

_N = 10000
_E = 320000
_DN = 128
_DE = 16
_F = 64

_NC = 2
_NS = 16
_NW = _NC * _NS
_C = 80
_CPW = _E // (_NS * _C)
_SEG = 5
_SEGR = _CPW // _SEG
_NPAD = 10240
_RPS = _NPAD // _NS


def _make_sc_spmm(gather):
  mesh = plsc.VectorSubcoreMesh(core_axis_name="c", subcore_axis_name="s")
  out_type = [jax.ShapeDtypeStruct((_NC, _NPAD, _DN), jnp.float32)]
  scratch = [
      pltpu.VMEM_SHARED((_NPAD, _DN), jnp.float32),
      pltpu.VMEM((_SEGR, _C), jnp.int32),
      pltpu.VMEM((2, _C, _DN), jnp.float32),
      pltpu.SemaphoreType.DMA,
      pltpu.SemaphoreType.DMA,
  ]
  if gather:
    scratch.append(pltpu.VMEM((_SEGR, _C), jnp.int32))

  def body(*refs):
    if gather:
      (x_hbm, srcr, dstr, zeros_hbm,
       out_hbm, acc, dst_idx, rows, sem0, sem1, src_idx) = refs
    else:
      (x_hbm, dstr, zeros_hbm,
       out_hbm, acc, dst_idx, rows, sem0, sem1) = refs
    cid = lax.axis_index("c")
    sid = lax.axis_index("s")
    pltpu.sync_copy(zeros_hbm, acc.at[pl.ds(sid * _RPS, _RPS)])
    plsc.subcore_barrier()
    wbase = (cid * _NS + sid) * _CPW * _C
    sems = (sem0, sem1)

    def segment(s, carry):
      pltpu.sync_copy(dstr.at[cid, sid, s], dst_idx)
      if gather:
        pltpu.sync_copy(srcr.at[cid, sid, s], src_idx)
      sbase = wbase + s * _SEGR * _C

      def fetch(j, slot):
        if gather:
          return pltpu.async_copy(x_hbm.at[src_idx.at[j]], rows.at[slot],
                                  sems[slot])
        return pltpu.async_copy(x_hbm.at[pl.ds(sbase + j * _C, _C)],
                                rows.at[slot], sems[slot])

      def drain(slot):
        pltpu.make_async_copy(x_hbm.at[pl.ds(0, _C)], rows.at[slot],
                              sems[slot]).wait()

      fetch(0, 0)

      def pair(i, carry2):
        j0 = 2 * i
        fetch(j0 + 1, 1)
        drain(0)
        pltpu.sync_copy(rows.at[0], acc.at[dst_idx.at[j0]], add=True)

        @pl.when(j0 + 2 < _SEGR)
        def _():
          fetch(j0 + 2, 0)

        drain(1)
        pltpu.sync_copy(rows.at[1], acc.at[dst_idx.at[j0 + 1]], add=True)
        return carry2

      lax.fori_loop(0, _SEGR // 2, pair, 0)
      return carry

    lax.fori_loop(0, _SEG, segment, 0)
    plsc.subcore_barrier()
    sl = pl.ds(sid * _RPS, _RPS)
    pltpu.sync_copy(acc.at[sl], out_hbm.at[cid].at[sl])

  return pl.kernel(body, out_type=out_type, mesh=mesh, scratch_types=scratch)


def _pre_body(e1, e2, wn1, mu1, mu2, xwcat, acc):
  pi = pl.program_id(0)

  @pl.when(pi == 0)
  def _():
    acc[...] = jnp.zeros_like(acc)

  acc[0:1, :] += jnp.sum(e1[...], axis=0, keepdims=True)
  acc[1:2, :] += jnp.sum(e2[...], axis=0, keepdims=True)
  xwcat[...] = jnp.concatenate(
      [jnp.dot(e1[...], wn1[...], preferred_element_type=jnp.float32),
       jnp.dot(e2[...], wn1[...], preferred_element_type=jnp.float32)],
      axis=-1)

  @pl.when(pi == pl.num_programs(0) - 1)
  def _():
    mu1[...] = acc[0:1, :] * (1.0 / _N)
    mu2[...] = acc[1:2, :] * (1.0 / _N)


def _ew_body(ea, we12, out):
  out[...] = jnp.dot(ea[...], we12[...], preferred_element_type=jnp.float32)


def _make_conv_body(tw_off, with_next):
  def body(*refs):
    if with_next:
      (sp, tp, x1, x2, mu1, mu2, ws, wc, wnext,
       h1, h2, sh1, sh2, hwcat, hacc) = refs
    else:
      (sp, tp, x1, x2, mu1, mu2, ws, wc,
       h1, h2, sh1, sh2, hacc) = refs
    pi = pl.program_id(0)

    def one(g, x, mu_other):
      sp_off = g * _F
      agg = (sp[g, :, sp_off:sp_off + _F] +
             tp[g, :, tw_off:tw_off + _F])
      agg += jnp.dot(x[...], ws[...], preferred_element_type=jnp.float32)
      ctx = jnp.dot(mu_other, wc[...], preferred_element_type=jnp.float32)
      return jnp.maximum(agg + ctx, 0.0)

    o1 = one(0, x1, mu2[...])
    o2 = one(1, x2, mu1[...])
    h1[...] = o1
    h2[...] = o2
    if with_next:
      hwcat[...] = jnp.concatenate(
          [jnp.dot(o1, wnext[...], preferred_element_type=jnp.float32),
           jnp.dot(o2, wnext[...], preferred_element_type=jnp.float32)],
          axis=-1)

    @pl.when(pi == 0)
    def _():
      hacc[...] = jnp.zeros_like(hacc)

    hacc[0:1, :] += jnp.sum(o1, axis=0, keepdims=True)
    hacc[1:2, :] += jnp.sum(o2, axis=0, keepdims=True)

    @pl.when(pi == pl.num_programs(0) - 1)
    def _():
      sh1[...] = hacc[0:1, :] * (1.0 / _N)
      sh2[...] = hacc[1:2, :] * (1.0 / _N)

  return body


def _match_body(h1, g1, h2t, g2t, mm1, mm2, mc1, mc2, avg,
                mapping, score, pre_ged, sacc):
  pi = pl.program_id(0)
  f32 = jnp.float32
  a = jnp.dot(h1[...], mm1[...], preferred_element_type=f32)
  b = jnp.dot(g1[...], mm2[...], preferred_element_type=f32)
  logits = (jnp.dot(a, h2t[...], preferred_element_type=f32) +
            jnp.dot(b, g2t[...], preferred_element_type=f32))
  m = jnp.max(logits, axis=1, keepdims=True)
  p = jnp.exp(logits - m)
  s = jnp.sum(p, axis=1, keepdims=True)
  mp = p / s
  mapping[...] = mp
  c = jnp.dot(h1[...], mc1[...], preferred_element_type=f32)
  d = jnp.dot(g1[...], mc2[...], preferred_element_type=f32)
  cost = (jnp.dot(c, h2t[...], preferred_element_type=f32) +
          jnp.dot(d, g2t[...], preferred_element_type=f32))
  partial = jnp.sum(mp * cost)
  prev = jnp.where(pi == 0, 0.0, sacc[0])
  tot = prev + partial
  sacc[0] = tot

  @pl.when(pi == pl.num_programs(0) - 1)
  def _():
    sc = 1.0 / (1.0 + jnp.exp(-tot))
    score[0] = sc
    pre_ged[0] = -jnp.log(sc) * avg[0]


@functools.lru_cache(maxsize=None)
def _get_sc_spmm(gather):
  return _make_sc_spmm(gather)


def _sc_spmm_call(x, src_both, dst_both):
  z = jnp.zeros((_RPS, _DN), jnp.float32)
  out, = _get_sc_spmm(True)(x, src_both, dst_both, z)
  return out


def _sc_easum_call(ewcat, dst_both):
  z = jnp.zeros((_RPS, _DN), jnp.float32)
  out, = _get_sc_spmm(False)(ewcat, dst_both, z)
  return out


def kernel(emb1, edge_index1, edge_attr1, emb2, edge_index2, edge_attr2,
           avg_v, Wn1, We1, Ws1, Wc1, Wn2, We2, Ws2, Wc2,
           Mmap1, Mmap2, Mcost1, Mcost2):
  f32 = jnp.float32
  bm = 1000
  grid = _N // bm

  mu1, mu2, xwcat = pl.pallas_call(
      _pre_body,
      grid=(grid,),
      in_specs=[pl.BlockSpec((bm, _DN), lambda i: (i, 0)),
                pl.BlockSpec((bm, _DN), lambda i: (i, 0)),
                pl.BlockSpec((_DN, _F), lambda i: (0, 0))],
      out_specs=[pl.BlockSpec((1, _DN), lambda i: (0, 0)),
                 pl.BlockSpec((1, _DN), lambda i: (0, 0)),
                 pl.BlockSpec((bm, _DN), lambda i: (i, 0))],
      out_shape=[jax.ShapeDtypeStruct((1, _DN), f32),
                 jax.ShapeDtypeStruct((1, _DN), f32),
                 jax.ShapeDtypeStruct((_N, _DN), f32)],
      scratch_shapes=[pltpu.VMEM((2, _DN), f32)],
  )(emb1, emb2, Wn1)

  src_both = jnp.stack([edge_index1[0].reshape(_NS, _SEG, _SEGR, _C),
                        edge_index2[0].reshape(_NS, _SEG, _SEGR, _C)])
  dst_both = jnp.stack([edge_index1[1].reshape(_NS, _SEG, _SEGR, _C),
                        edge_index2[1].reshape(_NS, _SEG, _SEGR, _C)])

  we12 = jnp.concatenate([We1, We2], axis=1)
  ea_cat = jnp.concatenate([edge_attr1, edge_attr2], axis=0)
  bme = 8000
  ewcat = pl.pallas_call(
      _ew_body,
      grid=(2 * _E // bme,),
      in_specs=[pl.BlockSpec((bme, _DE), lambda i: (i, 0)),
                pl.BlockSpec((_DE, _DN), lambda i: (0, 0))],
      out_specs=pl.BlockSpec((bme, _DN), lambda i: (i, 0)),
      out_shape=jax.ShapeDtypeStruct((2 * _E, _DN), f32),
  )(ea_cat, we12)

  sp = _sc_spmm_call(xwcat, src_both, dst_both)
  tp = _sc_easum_call(ewcat, dst_both)

  def conv_call(body, din, with_next, sp_, tp_, x1_, x2_,
                m1, m2, ws, wc, *wnext):
    return pl.pallas_call(
        body,
        grid=(grid,),
        in_specs=[
            pl.BlockSpec((_NC, bm, _DN), lambda i: (0, i, 0)),
            pl.BlockSpec((_NC, bm, _DN), lambda i: (0, i, 0)),
            pl.BlockSpec((bm, din), lambda i: (i, 0)),
            pl.BlockSpec((bm, din), lambda i: (i, 0)),
            pl.BlockSpec((1, din), lambda i: (0, 0)),
            pl.BlockSpec((1, din), lambda i: (0, 0)),
            pl.BlockSpec((din, _F), lambda i: (0, 0)),
            pl.BlockSpec((din, _F), lambda i: (0, 0)),
        ] + ([pl.BlockSpec((_F, _F), lambda i: (0, 0))] if with_next else []),
        out_specs=[pl.BlockSpec((bm, _F), lambda i: (i, 0)),
                   pl.BlockSpec((bm, _F), lambda i: (i, 0)),
                   pl.BlockSpec((1, _F), lambda i: (0, 0)),
                   pl.BlockSpec((1, _F), lambda i: (0, 0))] +
                  ([pl.BlockSpec((bm, _DN), lambda i: (i, 0))]
                   if with_next else []),
        out_shape=[jax.ShapeDtypeStruct((_N, _F), f32),
                   jax.ShapeDtypeStruct((_N, _F), f32),
                   jax.ShapeDtypeStruct((1, _F), f32),
                   jax.ShapeDtypeStruct((1, _F), f32)] +
                  ([jax.ShapeDtypeStruct((_N, _DN), f32)]
                   if with_next else []),
        scratch_shapes=[pltpu.VMEM((2, _F), f32)],
    )(sp_, tp_, x1_, x2_, m1, m2, ws, wc, *wnext)

  h1, h2, mh1, mh2, hwcat = conv_call(
      _make_conv_body(0, True), _DN, True,
      sp, tp, emb1, emb2, mu1, mu2, Ws1, Wc1, Wn2)

  up = _sc_spmm_call(hwcat, src_both, dst_both)

  g1, g2, _, _ = conv_call(
      _make_conv_body(_F, False), _F, False,
      up, tp, h1, h2, mh1, mh2, Ws2, Wc2)

  bm2 = 200
  grid2 = _N // bm2
  h2t = h2.T
  g2t = g2.T
  mapping, score, pre_ged = pl.pallas_call(
      _match_body,
      grid=(grid2,),
      in_specs=[
          pl.BlockSpec((bm2, _F), lambda i: (i, 0)),
          pl.BlockSpec((bm2, _F), lambda i: (i, 0)),
          pl.BlockSpec((_F, _N), lambda i: (0, 0)),
          pl.BlockSpec((_F, _N), lambda i: (0, 0)),
          pl.BlockSpec((_F, _F), lambda i: (0, 0)),
          pl.BlockSpec((_F, _F), lambda i: (0, 0)),
          pl.BlockSpec((_F, _F), lambda i: (0, 0)),
          pl.BlockSpec((_F, _F), lambda i: (0, 0)),
          pl.BlockSpec(memory_space=pltpu.SMEM),
      ],
      out_specs=[
          pl.BlockSpec((bm2, _N), lambda i: (i, 0)),
          pl.BlockSpec(memory_space=pltpu.SMEM),
          pl.BlockSpec(memory_space=pltpu.SMEM),
      ],
      out_shape=[jax.ShapeDtypeStruct((_N, _N), f32),
                 jax.ShapeDtypeStruct((1,), f32),
                 jax.ShapeDtypeStruct((1,), f32)],
      scratch_shapes=[pltpu.SMEM((1,), f32)],
  )(h1, g1, h2t, g2t, Mmap1, Mmap2, Mcost1, Mcost2, avg_v)

  return score, pre_ged, mapping
